# baseline (speedup 1.0000x reference)
import functools

import jax
import jax.numpy as jnp
from jax import lax
from jax.experimental import pallas as pl
from jax.experimental.pallas import tpu as pltpu
from jax.experimental.pallas import tpu_sc as plsc

NU, NM, E, D, H = 50000, 10000, 800000, 128, 64
EP = 819200
ROWS = EP // 128
NMP = 10112
NUP = 50048
UH = 25088
BLK = 1000


def _agg_call(mode):
  acc_rows = NMP if mode == 'edges' else UH
  nslots = 8 if mode == 'edges' else 3
  mesh = plsc.VectorSubcoreMesh(core_axis_name="c", subcore_axis_name="s")

  @functools.partial(
      pl.kernel,
      out_type=jax.ShapeDtypeStruct((2, acc_rows, 64), jnp.float32),
      mesh=mesh,
      scratch_types=[
          pltpu.VMEM((2, 8, 128), jnp.int32),
          pltpu.VMEM((2, 8, 128), jnp.int32),
          pltpu.VMEM((8, 128), jnp.int32),
          pltpu.VMEM((nslots, 128, 64), jnp.float32),
          pltpu.VMEM_SHARED((acc_rows, 64), jnp.float32),
          pltpu.SemaphoreType.DMA((nslots,)),
          pltpu.SemaphoreType.DMA((nslots,)),
          pltpu.SemaphoreType.DMA,
      ],
      compiler_params=pltpu.CompilerParams(use_tc_tiling_on_sc=False),
  )
  def k(x_hbm, gidx_hbm, sidx_hbm, z_hbm, out_hbm, gi2, si2, msi, rows_v, acc,
        gsem, ssem, isem):
    c = lax.axis_index("c")
    s = lax.axis_index("s")
    zr = acc_rows // 16
    pltpu.sync_copy(z_hbm.at[pl.ds(0, zr)], acc.at[pl.ds(s * zr, zr)])
    plsc.subcore_barrier()

    if mode == 'edges':
      base = (s * 2 + c) * 200
      nblk = 25
    else:
      base = s * 400
      nblk = 50
      lo = c * 25000

    def wait_g(slot):
      pltpu.make_async_copy(x_hbm.at[pl.ds(0, 128)], rows_v.at[slot],
                            gsem.at[slot]).wait()

    def wait_s(slot):
      pltpu.make_async_copy(rows_v.at[slot], acc.at[pl.ds(0, 128)],
                            ssem.at[slot]).wait()

    pltpu.sync_copy(gidx_hbm.at[pl.ds(base, 8)], gi2.at[0])
    pltpu.sync_copy(sidx_hbm.at[pl.ds(base, 8)], si2.at[0])

    def block(b, _):
      bi = b % 2
      r0 = base + b * 8

      @pl.when(b > 0)
      def _():
        pltpu.make_async_copy(gidx_hbm.at[pl.ds(base, 8)], gi2.at[bi],
                              isem).wait()
        pltpu.make_async_copy(sidx_hbm.at[pl.ds(base, 8)], si2.at[bi],
                              isem).wait()
        for t in range(nslots):
          wait_s(t)

      @pl.when(b + 1 < nblk)
      def _():
        nbi = (b + 1) % 2
        pltpu.async_copy(gidx_hbm.at[pl.ds(r0 + 8, 8)], gi2.at[nbi], isem)
        pltpu.async_copy(sidx_hbm.at[pl.ds(r0 + 8, 8)], si2.at[nbi], isem)

      if mode == 'nodes':
        for j in range(8):
          for i in range(8):
            v = si2[bi, j, pl.ds(i * 16, 16)]
            inr = (v >= lo) & (v < lo + 25000)
            msi[j, pl.ds(i * 16, 16)] = jnp.where(inr, v - lo, 25000)

      def sidx_row(r):
        return msi.at[r] if mode == 'nodes' else si2.at[bi].at[r]

      for t in range(min(nslots, 8)):
        pltpu.async_copy(x_hbm.at[gi2.at[bi].at[t]], rows_v.at[t],
                         gsem.at[t])
      for r in range(8):
        slot = r % nslots
        wait_g(slot)
        pltpu.async_copy(rows_v.at[slot], acc.at[sidx_row(r)],
                         ssem.at[slot], add=True)
        if r + nslots < 8:
          wait_s(slot)
          pltpu.async_copy(x_hbm.at[gi2.at[bi].at[r + nslots]],
                           rows_v.at[slot], gsem.at[slot])
      return _

    lax.fori_loop(0, nblk, block, None)
    for t in range(nslots):
      wait_s(t)
    plsc.subcore_barrier()
    wr = acc_rows // 16
    pltpu.sync_copy(acc.at[pl.ds(s * wr, wr)],
                    out_hbm.at[c].at[pl.ds(s * wr, wr)])

  return k


_agg_edges = _agg_call('edges')
_agg_nodes = _agg_call('nodes')


def _deg_kernel():
  mesh = plsc.VectorSubcoreMesh(core_axis_name="c", subcore_axis_name="s")

  @functools.partial(
      pl.kernel,
      out_type=(jax.ShapeDtypeStruct((NMP, 16), jnp.float32),
                jax.ShapeDtypeStruct((NUP, 16), jnp.float32)),
      mesh=mesh,
      scratch_types=[
          pltpu.VMEM((8, 128), jnp.int32),
          pltpu.VMEM((128, 16), jnp.float32),
          pltpu.VMEM_SHARED((NUP, 16), jnp.float32),
          pltpu.SemaphoreType.DMA,
      ],
      compiler_params=pltpu.CompilerParams(use_tc_tiling_on_sc=False),
  )
  def k(mdst_hbm, udst_hbm, z_hbm, ones_hbm, degm_hbm, degu_hbm, si_v, ones_v,
        acc, ssem):
    c = lax.axis_index("c")
    s = lax.axis_index("s")
    pltpu.sync_copy(ones_hbm, ones_v)
    acc_rows = jnp.where(c == 0, NMP, NUP)

    @pl.when(c == 0)
    def _():
      pltpu.sync_copy(z_hbm.at[pl.ds(0, NMP // 16)],
                      acc.at[pl.ds(s * (NMP // 16), NMP // 16)])

    @pl.when(c == 1)
    def _():
      pltpu.sync_copy(z_hbm.at[pl.ds(0, NUP // 16)],
                      acc.at[pl.ds(s * (NUP // 16), NUP // 16)])

    plsc.subcore_barrier()
    base = s * 400

    def chunk_m(ci, _):
      pltpu.sync_copy(mdst_hbm.at[pl.ds(base + ci * 8, 8)], si_v)
      sd = [pltpu.async_copy(ones_v, acc.at[si_v.at[j]], ssem, add=True)
            for j in range(8)]
      for d in sd:
        d.wait()
      return _

    def chunk_u(ci, _):
      pltpu.sync_copy(udst_hbm.at[pl.ds(base + ci * 8, 8)], si_v)
      sd = [pltpu.async_copy(ones_v, acc.at[si_v.at[j]], ssem, add=True)
            for j in range(8)]
      for d in sd:
        d.wait()
      return _

    @pl.when(c == 0)
    def _():
      lax.fori_loop(0, 50, chunk_m, None)

    @pl.when(c == 1)
    def _():
      lax.fori_loop(0, 50, chunk_u, None)

    plsc.subcore_barrier()

    @pl.when(c == 0)
    def _():
      wr = NMP // 16
      pltpu.sync_copy(acc.at[pl.ds(s * wr, wr)], degm_hbm.at[pl.ds(s * wr, wr)])

    @pl.when(c == 1)
    def _():
      wr = NUP // 16
      pltpu.sync_copy(acc.at[pl.ds(s * wr, wr)], degu_hbm.at[pl.ds(s * wr, wr)])

  return k


_deg = _deg_kernel()


def _proj(x, w, b):
  n = x.shape[0]
  grid = n // BLK

  def body(x_ref, w_ref, b_ref, o_ref):
    o_ref[...] = jnp.dot(x_ref[...], w_ref[...],
                         preferred_element_type=jnp.float32) + b_ref[...]

  return pl.pallas_call(
      body,
      grid=(grid,),
      in_specs=[
          pl.BlockSpec((BLK, D), lambda i: (i, 0)),
          pl.BlockSpec((D, H), lambda i: (0, 0)),
          pl.BlockSpec((1, H), lambda i: (0, 0)),
      ],
      out_specs=pl.BlockSpec((BLK, H), lambda i: (i, 0)),
      out_shape=jax.ShapeDtypeStruct((n, H), jnp.float32),
  )(x, w, b.reshape(1, H))


def _combine(agg, rdeg, x, wl, bl, wr, mode):
  n = x.shape[0]
  grid = n // BLK
  if mode == 'sum2':
    agg_spec = pl.BlockSpec((2, BLK, H), lambda i: (0, i, 0))
  else:
    agg_spec = pl.BlockSpec((1, BLK, H), lambda i: (i // 25, i % 25, 0))

  def body(agg_ref, rdeg_ref, x_ref, wl_ref, bl_ref, wr_ref, o_ref):
    a = jnp.sum(agg_ref[...], axis=0) * rdeg_ref[...]
    o_ref[...] = jax.nn.relu(
        jnp.dot(a, wl_ref[...], preferred_element_type=jnp.float32)
        + bl_ref[...]
        + jnp.dot(x_ref[...], wr_ref[...], preferred_element_type=jnp.float32))

  return pl.pallas_call(
      body,
      grid=(grid,),
      in_specs=[
          agg_spec,
          pl.BlockSpec((BLK, 1), lambda i: (i, 0)),
          pl.BlockSpec((BLK, H), lambda i: (i, 0)),
          pl.BlockSpec((H, H), lambda i: (0, 0)),
          pl.BlockSpec((1, H), lambda i: (0, 0)),
          pl.BlockSpec((H, H), lambda i: (0, 0)),
      ],
      out_specs=pl.BlockSpec((BLK, H), lambda i: (i, 0)),
      out_shape=jax.ShapeDtypeStruct((n, H), jnp.float32),
  )(agg, rdeg, x, wl, bl.reshape(1, H), wr)


def kernel(x_user, x_movie, edge_src, edge_dst, Wu, bu, Wm, bm,
           l0_u2m_Wl, l0_u2m_bl, l0_u2m_Wr, l0_m2u_Wl, l0_m2u_bl, l0_m2u_Wr,
           l1_u2m_Wl, l1_u2m_bl, l1_u2m_Wr, l1_m2u_Wl, l1_m2u_bl, l1_m2u_Wr):
  pad = EP - E
  es_s = jnp.concatenate([edge_src, jnp.full((pad,), NU, jnp.int32)])
  ed_s = jnp.concatenate([edge_dst, jnp.full((pad,), NM, jnp.int32)])
  es_g = jnp.concatenate([edge_src, jnp.zeros((pad,), jnp.int32)])
  ed_g = jnp.concatenate([edge_dst, jnp.zeros((pad,), jnp.int32)])
  es_s, ed_s = es_s.reshape(ROWS, 128), ed_s.reshape(ROWS, 128)
  es_g, ed_g = es_g.reshape(ROWS, 128), ed_g.reshape(ROWS, 128)
  z64 = jnp.zeros((UH // 16, 64), jnp.float32)
  z16 = jnp.zeros((NUP // 16, 16), jnp.float32)
  ones16 = jnp.ones((128, 16), jnp.float32)

  deg_m, deg_u = _deg(ed_s, es_s, z16, ones16)
  rdeg_m = 1.0 / jnp.maximum(deg_m[:, 0:1], 1.0)
  du = deg_u[:, 0:1]
  rdeg_u = 1.0 / jnp.maximum(du[:NU], 1.0)

  xu = _proj(x_user, Wu, bu)
  xm = _proj(x_movie, Wm, bm)

  layers = [
      (l0_u2m_Wl, l0_u2m_bl, l0_u2m_Wr, l0_m2u_Wl, l0_m2u_bl, l0_m2u_Wr),
      (l1_u2m_Wl, l1_u2m_bl, l1_u2m_Wr, l1_m2u_Wl, l1_m2u_bl, l1_m2u_Wr),
  ]
  for (aWl, abl, aWr, bWl, bbl, bWr) in layers:
    agg_m = _agg_edges(xu, es_g, ed_s, z64)
    agg_u = _agg_nodes(xm, ed_g, es_s, z64)
    new_m = _combine(agg_m, rdeg_m[:NM], xm, aWl, abl, aWr, 'sum2')
    new_u = _combine(agg_u, rdeg_u, xu, bWl, bbl, bWr, 'half')
    xm = new_m
    xu = new_u
  return (xu, xm)


# speedup vs baseline: 1.7929x; 1.7929x over previous
import functools

import jax
import jax.numpy as jnp
from jax import lax
from jax.experimental import pallas as pl
from jax.experimental.pallas import tpu as pltpu
from jax.experimental.pallas import tpu_sc as plsc

NU, NM, E, D, H = 50000, 10000, 800000, 128, 64
EP = 819200
ROWS = EP // 128
NMP = 10112
NUP = 50048
UH = 25088
BLK = 1000


def _agg_call(mode):
  acc_rows = NMP if mode == 'edges' else UH
  nslots = 8
  mesh = plsc.VectorSubcoreMesh(core_axis_name="c", subcore_axis_name="s")

  @functools.partial(
      pl.kernel,
      out_type=jax.ShapeDtypeStruct((2, acc_rows, 64), jnp.bfloat16),
      mesh=mesh,
      scratch_types=[
          pltpu.VMEM((2, 8, 128), jnp.int32),
          pltpu.VMEM((2, 8, 128), jnp.int32),
          pltpu.VMEM((8, 128), jnp.int32),
          pltpu.VMEM((nslots, 128, 64), jnp.bfloat16),
          pltpu.VMEM_SHARED((acc_rows, 64), jnp.bfloat16),
          pltpu.SemaphoreType.DMA((nslots,)),
          pltpu.SemaphoreType.DMA((nslots,)),
          pltpu.SemaphoreType.DMA,
      ],
      compiler_params=pltpu.CompilerParams(use_tc_tiling_on_sc=False),
  )
  def k(x_hbm, gidx_hbm, sidx_hbm, z_hbm, out_hbm, gi2, si2, msi, rows_v, acc,
        gsem, ssem, isem):
    c = lax.axis_index("c")
    s = lax.axis_index("s")
    zr = acc_rows // 16
    pltpu.sync_copy(z_hbm.at[pl.ds(0, zr)], acc.at[pl.ds(s * zr, zr)])
    plsc.subcore_barrier()

    if mode == 'edges':
      base = (s * 2 + c) * 200
      nblk = 25
    else:
      base = s * 400
      nblk = 50
      lo = c * 25000

    def wait_g(slot):
      pltpu.make_async_copy(x_hbm.at[pl.ds(0, 128)], rows_v.at[slot],
                            gsem.at[slot]).wait()

    def wait_s(slot):
      pltpu.make_async_copy(rows_v.at[slot], acc.at[pl.ds(0, 128)],
                            ssem.at[slot]).wait()

    pltpu.sync_copy(gidx_hbm.at[pl.ds(base, 8)], gi2.at[0])
    pltpu.sync_copy(sidx_hbm.at[pl.ds(base, 8)], si2.at[0])

    def block(b, _):
      bi = b % 2
      r0 = base + b * 8

      @pl.when(b > 0)
      def _():
        pltpu.make_async_copy(gidx_hbm.at[pl.ds(base, 8)], gi2.at[bi],
                              isem).wait()
        pltpu.make_async_copy(sidx_hbm.at[pl.ds(base, 8)], si2.at[bi],
                              isem).wait()
        for t in range(nslots):
          wait_s(t)

      @pl.when(b + 1 < nblk)
      def _():
        nbi = (b + 1) % 2
        pltpu.async_copy(gidx_hbm.at[pl.ds(r0 + 8, 8)], gi2.at[nbi], isem)
        pltpu.async_copy(sidx_hbm.at[pl.ds(r0 + 8, 8)], si2.at[nbi], isem)

      if mode == 'nodes':
        for j in range(8):
          for i in range(8):
            v = si2[bi, j, pl.ds(i * 16, 16)]
            inr = (v >= lo) & (v < lo + 25000)
            msi[j, pl.ds(i * 16, 16)] = jnp.where(inr, v - lo, 25000)

      def sidx_row(r):
        return msi.at[r] if mode == 'nodes' else si2.at[bi].at[r]

      for t in range(min(nslots, 8)):
        pltpu.async_copy(x_hbm.at[gi2.at[bi].at[t]], rows_v.at[t],
                         gsem.at[t])
      for r in range(8):
        slot = r % nslots
        wait_g(slot)
        pltpu.async_copy(rows_v.at[slot], acc.at[sidx_row(r)],
                         ssem.at[slot], add=True)
        if r + nslots < 8:
          wait_s(slot)
          pltpu.async_copy(x_hbm.at[gi2.at[bi].at[r + nslots]],
                           rows_v.at[slot], gsem.at[slot])
      return _

    lax.fori_loop(0, nblk, block, None)
    for t in range(nslots):
      wait_s(t)
    plsc.subcore_barrier()
    wr = acc_rows // 16
    pltpu.sync_copy(acc.at[pl.ds(s * wr, wr)],
                    out_hbm.at[c].at[pl.ds(s * wr, wr)])

  return k


_agg_edges = _agg_call('edges')
_agg_nodes = _agg_call('nodes')


def _deg_kernel():
  mesh = plsc.VectorSubcoreMesh(core_axis_name="c", subcore_axis_name="s")

  @functools.partial(
      pl.kernel,
      out_type=(jax.ShapeDtypeStruct((NMP, 16), jnp.float32),
                jax.ShapeDtypeStruct((NUP, 16), jnp.float32)),
      mesh=mesh,
      scratch_types=[
          pltpu.VMEM((8, 128), jnp.int32),
          pltpu.VMEM((128, 16), jnp.float32),
          pltpu.VMEM_SHARED((NUP, 16), jnp.float32),
          pltpu.SemaphoreType.DMA,
      ],
      compiler_params=pltpu.CompilerParams(use_tc_tiling_on_sc=False),
  )
  def k(mdst_hbm, udst_hbm, z_hbm, ones_hbm, degm_hbm, degu_hbm, si_v, ones_v,
        acc, ssem):
    c = lax.axis_index("c")
    s = lax.axis_index("s")
    pltpu.sync_copy(ones_hbm, ones_v)
    acc_rows = jnp.where(c == 0, NMP, NUP)

    @pl.when(c == 0)
    def _():
      pltpu.sync_copy(z_hbm.at[pl.ds(0, NMP // 16)],
                      acc.at[pl.ds(s * (NMP // 16), NMP // 16)])

    @pl.when(c == 1)
    def _():
      pltpu.sync_copy(z_hbm.at[pl.ds(0, NUP // 16)],
                      acc.at[pl.ds(s * (NUP // 16), NUP // 16)])

    plsc.subcore_barrier()
    base = s * 400

    def chunk_m(ci, _):
      pltpu.sync_copy(mdst_hbm.at[pl.ds(base + ci * 8, 8)], si_v)
      sd = [pltpu.async_copy(ones_v, acc.at[si_v.at[j]], ssem, add=True)
            for j in range(8)]
      for d in sd:
        d.wait()
      return _

    def chunk_u(ci, _):
      pltpu.sync_copy(udst_hbm.at[pl.ds(base + ci * 8, 8)], si_v)
      sd = [pltpu.async_copy(ones_v, acc.at[si_v.at[j]], ssem, add=True)
            for j in range(8)]
      for d in sd:
        d.wait()
      return _

    @pl.when(c == 0)
    def _():
      lax.fori_loop(0, 50, chunk_m, None)

    @pl.when(c == 1)
    def _():
      lax.fori_loop(0, 50, chunk_u, None)

    plsc.subcore_barrier()

    @pl.when(c == 0)
    def _():
      wr = NMP // 16
      pltpu.sync_copy(acc.at[pl.ds(s * wr, wr)], degm_hbm.at[pl.ds(s * wr, wr)])

    @pl.when(c == 1)
    def _():
      wr = NUP // 16
      pltpu.sync_copy(acc.at[pl.ds(s * wr, wr)], degu_hbm.at[pl.ds(s * wr, wr)])

  return k


_deg = _deg_kernel()


def _proj(x, w, b):
  n = x.shape[0]
  grid = n // BLK

  def body(x_ref, w_ref, b_ref, o_ref, o16_ref):
    y = jnp.dot(x_ref[...], w_ref[...],
                preferred_element_type=jnp.float32) + b_ref[...]
    o_ref[...] = y
    o16_ref[...] = y.astype(jnp.bfloat16)

  return pl.pallas_call(
      body,
      grid=(grid,),
      in_specs=[
          pl.BlockSpec((BLK, D), lambda i: (i, 0)),
          pl.BlockSpec((D, H), lambda i: (0, 0)),
          pl.BlockSpec((1, H), lambda i: (0, 0)),
      ],
      out_specs=[pl.BlockSpec((BLK, H), lambda i: (i, 0)),
                 pl.BlockSpec((BLK, H), lambda i: (i, 0))],
      out_shape=[jax.ShapeDtypeStruct((n, H), jnp.float32),
                 jax.ShapeDtypeStruct((n, H), jnp.bfloat16)],
  )(x, w, b.reshape(1, H))


def _combine(agg, rdeg, x, wl, bl, wr, mode):
  n = x.shape[0]
  grid = n // BLK
  if mode == 'sum2':
    agg_spec = pl.BlockSpec((2, BLK, H), lambda i: (0, i, 0))
  else:
    agg_spec = pl.BlockSpec((1, BLK, H), lambda i: (i // 25, i % 25, 0))

  def body(agg_ref, rdeg_ref, x_ref, wl_ref, bl_ref, wr_ref, o_ref, o16_ref):
    a = jnp.sum(agg_ref[...].astype(jnp.float32), axis=0) * rdeg_ref[...]
    y = jax.nn.relu(
        jnp.dot(a, wl_ref[...], preferred_element_type=jnp.float32)
        + bl_ref[...]
        + jnp.dot(x_ref[...], wr_ref[...], preferred_element_type=jnp.float32))
    o_ref[...] = y
    o16_ref[...] = y.astype(jnp.bfloat16)

  return pl.pallas_call(
      body,
      grid=(grid,),
      in_specs=[
          agg_spec,
          pl.BlockSpec((BLK, 1), lambda i: (i, 0)),
          pl.BlockSpec((BLK, H), lambda i: (i, 0)),
          pl.BlockSpec((H, H), lambda i: (0, 0)),
          pl.BlockSpec((1, H), lambda i: (0, 0)),
          pl.BlockSpec((H, H), lambda i: (0, 0)),
      ],
      out_specs=[pl.BlockSpec((BLK, H), lambda i: (i, 0)),
                 pl.BlockSpec((BLK, H), lambda i: (i, 0))],
      out_shape=[jax.ShapeDtypeStruct((n, H), jnp.float32),
                 jax.ShapeDtypeStruct((n, H), jnp.bfloat16)],
  )(agg, rdeg, x, wl, bl.reshape(1, H), wr)


def kernel(x_user, x_movie, edge_src, edge_dst, Wu, bu, Wm, bm,
           l0_u2m_Wl, l0_u2m_bl, l0_u2m_Wr, l0_m2u_Wl, l0_m2u_bl, l0_m2u_Wr,
           l1_u2m_Wl, l1_u2m_bl, l1_u2m_Wr, l1_m2u_Wl, l1_m2u_bl, l1_m2u_Wr):
  pad = EP - E
  es_s = jnp.concatenate([edge_src, jnp.full((pad,), NU, jnp.int32)])
  ed_s = jnp.concatenate([edge_dst, jnp.full((pad,), NM, jnp.int32)])
  es_g = jnp.concatenate([edge_src, jnp.zeros((pad,), jnp.int32)])
  ed_g = jnp.concatenate([edge_dst, jnp.zeros((pad,), jnp.int32)])
  es_s, ed_s = es_s.reshape(ROWS, 128), ed_s.reshape(ROWS, 128)
  es_g, ed_g = es_g.reshape(ROWS, 128), ed_g.reshape(ROWS, 128)
  z64 = jnp.zeros((UH // 16, 64), jnp.bfloat16)
  z16 = jnp.zeros((NUP // 16, 16), jnp.float32)
  ones16 = jnp.ones((128, 16), jnp.float32)

  deg_m, deg_u = _deg(ed_s, es_s, z16, ones16)
  rdeg_m = 1.0 / jnp.maximum(deg_m[:, 0:1], 1.0)
  du = deg_u[:, 0:1]
  rdeg_u = 1.0 / jnp.maximum(du[:NU], 1.0)

  xu, xu16 = _proj(x_user, Wu, bu)
  xm, xm16 = _proj(x_movie, Wm, bm)

  layers = [
      (l0_u2m_Wl, l0_u2m_bl, l0_u2m_Wr, l0_m2u_Wl, l0_m2u_bl, l0_m2u_Wr),
      (l1_u2m_Wl, l1_u2m_bl, l1_u2m_Wr, l1_m2u_Wl, l1_m2u_bl, l1_m2u_Wr),
  ]
  for (aWl, abl, aWr, bWl, bbl, bWr) in layers:
    agg_m = _agg_edges(xu16, es_g, ed_s, z64)
    agg_u = _agg_nodes(xm16, ed_g, es_s, z64)
    xm, xm16 = _combine(agg_m, rdeg_m[:NM], xm, aWl, abl, aWr, 'sum2')
    xu, xu16 = _combine(agg_u, rdeg_u, xu, bWl, bbl, bWr, 'half')
  return (xu, xm)
